# baseline (speedup 1.0000x reference)
import functools
import jax
import jax.numpy as jnp
from jax import lax
from jax.experimental import pallas as pl
from jax.experimental.pallas import tpu as pltpu
from jax.experimental.pallas import tpu_sc as plsc

NN = 10000
EE = 160000
HH = 16
NTYPE = 50
NC, NS, LANES = 2, 16, 16
NW = NC * NS
CH = 768
EPW0 = 9 * CH
EPW1 = 5 * CH
EPAD = (EPW0 + EPW1) * NS
GW = 32
NPAD = 10240
NPW = NPAD // NW
NNP = NPAD * NTYPE
ROWS_PER_SUB = NPAD // NS

_f32 = jnp.float32


def _h0_gather_body(nid_h, q_h, out_h, idxv, rowsv, sem):
    cid = lax.axis_index("c")
    sid = lax.axis_index("s")
    wid = sid * NC + cid
    for c in range(NPW // 80):
        base = wid * NPW + c * 80
        pltpu.sync_copy(nid_h.at[pl.ds(base, 80)], idxv)
        pltpu.async_copy(q_h.at[idxv], rowsv, sem).wait()
        pltpu.sync_copy(rowsv, out_h.at[pl.ds(base, 80)])


_h0_gather = pl.kernel(
    _h0_gather_body,
    out_type=jax.ShapeDtypeStruct((NPAD, HH), _f32),
    compiler_params=pltpu.CompilerParams(use_tc_tiling_on_sc=False, needs_layout_passes=False),
    mesh=plsc.VectorSubcoreMesh(
        core_axis_name="c", subcore_axis_name="s", num_cores=NC,
        num_subcores=NS),
    scratch_types=[
        pltpu.VMEM((80,), jnp.int32),
        pltpu.VMEM((80, HH), _f32),
        pltpu.SemaphoreType.DMA,
    ],
)


def _edge_pass_body(dst_h, gidx_h, a2_h, g_h,
                    num_o, den_o,
                    dstv, gidxv, a2v, exv, gv, outv,
                    acc_num, acc_den, sem, ssem):
    cid = lax.axis_index("c")
    sid = lax.axis_index("s")

    zero16 = jnp.zeros((LANES,), _f32)
    for i in range(ROWS_PER_SUB // LANES):
        exv[0, pl.ds(i * LANES, LANES)] = zero16
    for i in range(ROWS_PER_SUB):
        outv[0, i] = zero16
    pltpu.sync_copy(outv.at[0, pl.ds(0, ROWS_PER_SUB)],
                    acc_num.at[pl.ds(sid * ROWS_PER_SUB, ROWS_PER_SUB)])
    pltpu.sync_copy(exv.at[0, pl.ds(0, ROWS_PER_SUB)],
                    acc_den.at[pl.ds(sid * ROWS_PER_SUB, ROWS_PER_SUB)])
    plsc.subcore_barrier()

    lane_iota = lax.iota(jnp.int32, LANES)
    col16 = jnp.full((LANES,), LANES, jnp.int32)

    def issue_gathers(ci, b):
        c0 = ci * CH
        return [
            pltpu.async_copy(a2_h.at[dstv.at[pl.ds(c0, CH)]], a2v.at[b], sem),
            pltpu.async_copy(g_h.at[gidxv.at[pl.ds(c0, CH)]], gv.at[b], sem),
        ]

    def compute(ci, b, ebase):
        base = ebase + ci * CH

        def group(gi, c):
            rowi = gi * LANES + lane_iota
            s = (plsc.load_gather(gv.at[b], [rowi, col16])
                 + plsc.load_gather(a2v.at[b], [rowi]))
            l = jnp.where(s >= 0.0, s, 0.2 * s)
            ex = jnp.exp(l)
            eid = base + rowi
            ex = jnp.where(eid < EE, ex, 0.0)
            plsc.store_scatter(exv.at[b], [rowi], ex)
            for j in range(HH):
                colj = jnp.full((LANES,), j, jnp.int32)
                colt = plsc.load_gather(gv.at[b], [rowi, colj])
                plsc.store_scatter(outv.at[b], [rowi, colj], colt * ex)
            return c

        lax.fori_loop(0, CH // LANES, group, 0)

    def issue_scatter(ci, b):
        c0 = ci * CH
        return [
            pltpu.async_copy(outv.at[b], acc_num.at[dstv.at[pl.ds(c0, CH)]],
                             ssem, add=True),
            pltpu.async_copy(exv.at[b], acc_den.at[dstv.at[pl.ds(c0, CH)]],
                             ssem, add=True),
        ]

    def run_pipeline(ebase, nchunk):
        epw = nchunk * CH
        pltpu.sync_copy(dst_h.at[pl.ds(ebase, epw)], dstv.at[pl.ds(0, epw)])
        pltpu.sync_copy(gidx_h.at[pl.ds(ebase, epw)], gidxv.at[pl.ds(0, epw)])
        gd = issue_gathers(0, 0)
        sd = [None, None]
        for ci in range(nchunk):
            b = ci % 2
            for d in gd:
                d.wait()
            if ci + 1 < nchunk:
                gd = issue_gathers(ci + 1, 1 - b)
            if sd[b] is not None:
                for d in sd[b]:
                    d.wait()
            compute(ci, b, ebase)
            sd[b] = issue_scatter(ci, b)
        for dl in sd:
            if dl is not None:
                for d in dl:
                    d.wait()

    @pl.when(cid == 0)
    def _():
        run_pipeline(sid * EPW0, EPW0 // CH)

    @pl.when(cid == 1)
    def _():
        run_pipeline(NS * EPW0 + sid * EPW1, EPW1 // CH)

    plsc.subcore_barrier()

    row0 = sid * ROWS_PER_SUB
    pltpu.sync_copy(acc_num.at[pl.ds(row0, ROWS_PER_SUB)],
                    num_o.at[cid, pl.ds(row0, ROWS_PER_SUB)])
    pltpu.sync_copy(acc_den.at[pl.ds(row0, ROWS_PER_SUB)],
                    den_o.at[cid, pl.ds(row0, ROWS_PER_SUB)])


_edge_pass = pl.kernel(
    _edge_pass_body,
    out_type=(
        jax.ShapeDtypeStruct((NC, NPAD, HH), _f32),
        jax.ShapeDtypeStruct((NC, NPAD), _f32),
    ),
    compiler_params=pltpu.CompilerParams(use_tc_tiling_on_sc=False, needs_layout_passes=False),
    mesh=plsc.VectorSubcoreMesh(
        core_axis_name="c", subcore_axis_name="s", num_cores=NC,
        num_subcores=NS),
    scratch_types=[
        pltpu.VMEM((EPW0,), jnp.int32),
        pltpu.VMEM((EPW0,), jnp.int32),
        pltpu.VMEM((2, CH), _f32),
        pltpu.VMEM((2, CH), _f32),
        pltpu.VMEM((2, CH, GW), _f32),
        pltpu.VMEM((2, CH, HH), _f32),
        pltpu.VMEM_SHARED((NPAD, HH), _f32),
        pltpu.VMEM_SHARED((NPAD,), _f32),
        pltpu.SemaphoreType.DMA,
        pltpu.SemaphoreType.DMA,
    ],
)


def _tables_body(ne_ref, pwt_ref, pb_ref, ee_ref,
                 w11_ref, b11_ref, w12_ref, b12_ref,
                 w21_ref, b21_ref, w22_ref, b22_ref,
                 q_ref, t_ref):
    q_ref[...] = jnp.maximum(
        jnp.dot(ne_ref[...], pwt_ref[...], preferred_element_type=_f32)
        + pb_ref[...], 0.0)
    ee = ee_ref[...]
    t1 = jnp.maximum(
        jnp.dot(ee, w11_ref[...], preferred_element_type=_f32) + b11_ref[...],
        0.0)
    t1 = jnp.dot(t1, w12_ref[...], preferred_element_type=_f32) + b12_ref[...]
    t2 = jnp.maximum(
        jnp.dot(ee, w21_ref[...], preferred_element_type=_f32) + b21_ref[...],
        0.0)
    t2 = jnp.dot(t2, w22_ref[...], preferred_element_type=_f32) + b22_ref[...]
    tid = lax.broadcasted_iota(jnp.int32, (NTYPE, 1), 0)
    t_ref[...] = jnp.where(tid < 4, t1, t2)


def _tc_tables(node_emb, proj_wt, proj_b, edge_emb,
               w11, b11, w12, b12, w21, b21, w22, b22):
    return pl.pallas_call(
        _tables_body,
        out_shape=(
            jax.ShapeDtypeStruct((NN, HH), _f32),
            jax.ShapeDtypeStruct((NTYPE, HH * HH), _f32),
        ),
    )(node_emb, proj_wt, proj_b, edge_emb,
      w11, b11, w12, b12, w21, b21, w22, b22)


def _gru(hn, h, gw):
    (wir, wiz, win, whr, whz, whn, bir, biz, bin_, bhr, bhz, bhn) = gw
    ir = jnp.dot(hn, wir, preferred_element_type=_f32) + bir
    iz = jnp.dot(hn, wiz, preferred_element_type=_f32) + biz
    inn = jnp.dot(hn, win, preferred_element_type=_f32) + bin_
    hr = jnp.dot(h, whr, preferred_element_type=_f32) + bhr
    hz = jnp.dot(h, whz, preferred_element_type=_f32) + bhz
    hnn = jnp.dot(h, whn, preferred_element_type=_f32) + bhn
    r = jax.nn.sigmoid(ir + hr)
    z = jax.nn.sigmoid(iz + hz)
    n = jnp.tanh(inn + r * hnn)
    return (1.0 - z) * n + z * h


def _prep_out(hnew, wbig_ref, aw2_ref, g_ref, a2_ref):
    g_ref[...] = jnp.dot(hnew, wbig_ref[...], preferred_element_type=_f32)
    a2_ref[...] = jnp.sum(hnew * aw2_ref[...], axis=1, keepdims=True)


NBLK = 10
BLK = NPAD // NBLK


def _step_body(num_ref, den_ref, h_ref, convb_ref,
               wir, wiz, win, whr, whz, whn, bir, biz, bin_, bhr, bhz, bhn,
               wbig_ref, aw2_ref,
               hnew_ref, g_ref, a2_ref):
    num = num_ref[0] + num_ref[1]
    den = den_ref[0] + den_ref[1]
    agg = num / (den + 1e-16) + convb_ref[...]
    hn = jnp.maximum(agg, 0.0)
    h = h_ref[...]
    gw = (wir[...], wiz[...], win[...], whr[...], whz[...], whn[...],
          bir[...], biz[...], bin_[...], bhr[...], bhz[...], bhn[...])
    hnew = _gru(hn, h, gw)
    hnew_ref[...] = hnew
    _prep_out(hnew, wbig_ref, aw2_ref, g_ref, a2_ref)


def _final_body(num_ref, den_ref, h_ref, convb_ref,
                wir, wiz, win, whr, whz, whn, bir, biz, bin_, bhr, bhz, bhn,
                hnew_ref):
    num = num_ref[0] + num_ref[1]
    den = den_ref[0] + den_ref[1]
    agg = num / (den + 1e-16) + convb_ref[...]
    hn = jnp.maximum(agg, 0.0)
    h = h_ref[...]
    gw = (wir[...], wiz[...], win[...], whr[...], whz[...], whn[...],
          bir[...], biz[...], bin_[...], bhr[...], bhz[...], bhn[...])
    hnew_ref[...] = _gru(hn, h, gw)


def _prep_body(h_ref, wbig_ref, aw2_ref, g_ref, a2_ref):
    _prep_out(h_ref[...], wbig_ref, aw2_ref, g_ref, a2_ref)


def _blk(shape):
    return pl.BlockSpec(shape, lambda i: (0,) * len(shape))


_num_spec = pl.BlockSpec((NC, BLK, HH), lambda i: (0, i, 0))
_den_spec = pl.BlockSpec((NC, BLK, 1), lambda i: (0, i, 0))
_h_spec = pl.BlockSpec((BLK, HH), lambda i: (i, 0))
_g_spec = pl.BlockSpec((BLK, NTYPE * GW), lambda i: (i, 0))
_a_spec = pl.BlockSpec((BLK, 1), lambda i: (i, 0))

_GW_SHAPES = [(HH, HH)] * 6 + [(1, HH)] * 6


def _tc_step(num, den, h, convb, gws, wbig, aw2):
    return pl.pallas_call(
        _step_body,
        grid=(NBLK,),
        in_specs=[_num_spec, _den_spec, _h_spec, _blk((1, HH))] +
                 [_blk(s) for s in _GW_SHAPES] +
                 [_blk((HH, NTYPE * GW)), _blk((1, HH))],
        out_specs=[_h_spec, _g_spec, _a_spec],
        out_shape=(
            jax.ShapeDtypeStruct((NPAD, HH), _f32),
            jax.ShapeDtypeStruct((NPAD, NTYPE * GW), _f32),
            jax.ShapeDtypeStruct((NPAD, 1), _f32),
        ),
    )(num, den, h, convb, *gws, wbig, aw2)


def _tc_final(num, den, h, convb, gws):
    return pl.pallas_call(
        _final_body,
        grid=(NBLK,),
        in_specs=[_num_spec, _den_spec, _h_spec, _blk((1, HH))] +
                 [_blk(s) for s in _GW_SHAPES],
        out_specs=[_h_spec],
        out_shape=(jax.ShapeDtypeStruct((NPAD, HH), _f32),),
    )(num, den, h, convb, *gws)[0]


def _tc_prep(h, wbig, aw2):
    return pl.pallas_call(
        _prep_body,
        grid=(NBLK,),
        in_specs=[_h_spec, _blk((HH, NTYPE * GW)), _blk((1, HH))],
        out_specs=[_g_spec, _a_spec],
        out_shape=(
            jax.ShapeDtypeStruct((NPAD, NTYPE * GW), _f32),
            jax.ShapeDtypeStruct((NPAD, 1), _f32),
        ),
    )(h, wbig, aw2)


@jax.jit
def kernel(node_ids, edge_ids, edge_index, node_emb, edge_emb, proj_w, proj_b,
           attn_w, e1_w1, e1_b1, e1_w2, e1_b2, e2_w1, e2_b1, e2_w2, e2_b2,
           conv_b, gru_w_ih, gru_w_hh, gru_b_ih, gru_b_hh):
    src = edge_index[0].astype(jnp.int32)
    dst = edge_index[1].astype(jnp.int32)
    et = edge_ids.astype(jnp.int32)
    pad = EPAD - EE
    dstp = jnp.concatenate([dst, jnp.zeros((pad,), jnp.int32)])
    gidxp = jnp.concatenate([src * NTYPE + et, jnp.zeros((pad,), jnp.int32)])
    nidp = jnp.concatenate(
        [node_ids.astype(jnp.int32), jnp.zeros((NPAD - NN,), jnp.int32)])

    convb = conv_b.reshape(1, HH)
    aw1 = attn_w[:, :HH].reshape(1, HH)
    aw2 = attn_w[:, HH:].reshape(1, HH)
    gws = (gru_w_ih[:HH].T, gru_w_ih[HH:2 * HH].T, gru_w_ih[2 * HH:].T,
           gru_w_hh[:HH].T, gru_w_hh[HH:2 * HH].T, gru_w_hh[2 * HH:].T,
           gru_b_ih[:HH].reshape(1, HH), gru_b_ih[HH:2 * HH].reshape(1, HH),
           gru_b_ih[2 * HH:].reshape(1, HH),
           gru_b_hh[:HH].reshape(1, HH), gru_b_hh[HH:2 * HH].reshape(1, HH),
           gru_b_hh[2 * HH:].reshape(1, HH))

    q, t = _tc_tables(node_emb, proj_w.T, proj_b.reshape(1, HH), edge_emb,
                      e1_w1.T, e1_b1.reshape(1, -1), e1_w2.T,
                      e1_b2.reshape(1, -1), e2_w1.T, e2_b1.reshape(1, -1),
                      e2_w2.T, e2_b2.reshape(1, -1))
    wcat3 = t.reshape(NTYPE, HH, HH).transpose(1, 0, 2)
    wbig = jnp.concatenate(
        [wcat3,
         jnp.broadcast_to(aw1.reshape(HH, 1, 1), (HH, NTYPE, 1)),
         jnp.zeros((HH, NTYPE, GW - HH - 1), _f32)],
        axis=2).reshape(HH, NTYPE * GW)

    h = _h0_gather(nidp, q)
    g, a2 = _tc_prep(h, wbig, aw2)

    for step in range(3):
        gflat = g.reshape(NPAD * NTYPE, GW)
        num_p, den_p = _edge_pass(dstp, gidxp, a2.reshape(NPAD), gflat)
        den = den_p.reshape(NC, NPAD, 1)
        if step < 2:
            h, g, a2 = _tc_step(num_p, den, h, convb, gws, wbig, aw2)
        else:
            h = _tc_final(num_p, den, h, convb, gws)
    return h[:NN]


# speedup vs baseline: 1.5702x; 1.5702x over previous
import jax
import jax.numpy as jnp
from jax import lax
from jax.experimental import pallas as pl
from jax.experimental.pallas import tpu as pltpu
from jax.experimental.pallas import tpu_sc as plsc

NN = 10000
EE = 160000
HH = 16
NTYPE = 50
NC, NS, LANES = 2, 16, 16
NW = NC * NS
CH = 1024
EPW0 = 6 * CH
EPW1 = 4 * CH
EPAD = (EPW0 + EPW1) * NS
NPAD = 10240
NPW = NPAD // NW
NNP = NPAD * NTYPE
ROWS_PER_SUB = NPAD // NS

_f32 = jnp.float32

_SC_PARAMS = pltpu.CompilerParams(use_tc_tiling_on_sc=False,
                                  needs_layout_passes=False)


def _h0_gather_body(nid_h, q_h, out_h, idxv, rowsv, sem):
    cid = lax.axis_index("c")
    sid = lax.axis_index("s")
    wid = sid * NC + cid
    for c in range(NPW // 80):
        base = wid * NPW + c * 80
        pltpu.sync_copy(nid_h.at[pl.ds(base, 80)], idxv)
        pltpu.async_copy(q_h.at[idxv], rowsv, sem).wait()
        pltpu.sync_copy(rowsv, out_h.at[pl.ds(base, 80)])


_h0_gather = pl.kernel(
    _h0_gather_body,
    out_type=jax.ShapeDtypeStruct((NPAD, HH), _f32),
    compiler_params=_SC_PARAMS,
    mesh=plsc.VectorSubcoreMesh(
        core_axis_name="c", subcore_axis_name="s", num_cores=NC,
        num_subcores=NS),
    scratch_types=[
        pltpu.VMEM((80,), jnp.int32),
        pltpu.VMEM((80, HH), _f32),
        pltpu.SemaphoreType.DMA,
    ],
)


def _edge_pass_body(src_h, dst_h, gidx_h, a1_h, a2_h, g_h,
                    num_o, den_o,
                    srcv, dstv, gidxv, a1v, a2v, exv, gv, outv,
                    acc_num, acc_den, sem, ssem):
    cid = lax.axis_index("c")
    sid = lax.axis_index("s")

    zero16 = jnp.zeros((LANES,), _f32)
    for i in range(ROWS_PER_SUB // LANES):
        exv[0, pl.ds(i * LANES, LANES)] = zero16
    for i in range(ROWS_PER_SUB):
        outv[0, i] = zero16
    pltpu.sync_copy(outv.at[0, pl.ds(0, ROWS_PER_SUB)],
                    acc_num.at[pl.ds(sid * ROWS_PER_SUB, ROWS_PER_SUB)])
    pltpu.sync_copy(exv.at[0, pl.ds(0, ROWS_PER_SUB)],
                    acc_den.at[pl.ds(sid * ROWS_PER_SUB, ROWS_PER_SUB)])
    plsc.subcore_barrier()

    lane_iota = lax.iota(jnp.int32, LANES)

    def issue_gathers(ci, b):
        c0 = ci * CH
        return [
            pltpu.async_copy(a1_h.at[srcv.at[pl.ds(c0, CH)]], a1v.at[b], sem),
            pltpu.async_copy(a2_h.at[dstv.at[pl.ds(c0, CH)]], a2v.at[b], sem),
            pltpu.async_copy(g_h.at[gidxv.at[pl.ds(c0, CH)]], gv.at[b], sem),
        ]

    def compute(ci, b, ebase):
        base = ebase + ci * CH

        def group(gi, c):
            rowi = gi * LANES + lane_iota
            s = (plsc.load_gather(a1v.at[b], [rowi])
                 + plsc.load_gather(a2v.at[b], [rowi]))
            l = jnp.where(s >= 0.0, s, 0.2 * s)
            ex = jnp.exp(l)
            eid = base + rowi
            ex = jnp.where(eid < EE, ex, 0.0)
            plsc.store_scatter(exv.at[b], [rowi], ex)
            for j in range(HH):
                colj = jnp.full((LANES,), j, jnp.int32)
                colt = plsc.load_gather(gv.at[b], [rowi, colj])
                plsc.store_scatter(outv.at[b], [rowi, colj], colt * ex)
            return c

        lax.fori_loop(0, CH // LANES, group, 0)

    def issue_scatters(ci, b):
        c0 = ci * CH
        return [
            pltpu.async_copy(outv.at[b], acc_num.at[dstv.at[pl.ds(c0, CH)]],
                             ssem, add=True),
            pltpu.async_copy(exv.at[b], acc_den.at[dstv.at[pl.ds(c0, CH)]],
                             ssem, add=True),
        ]

    def run_pipeline(ebase, nchunk):
        epw = nchunk * CH
        pltpu.sync_copy(src_h.at[pl.ds(ebase, epw)], srcv.at[pl.ds(0, epw)])
        pltpu.sync_copy(dst_h.at[pl.ds(ebase, epw)], dstv.at[pl.ds(0, epw)])
        pltpu.sync_copy(gidx_h.at[pl.ds(ebase, epw)], gidxv.at[pl.ds(0, epw)])
        gd = issue_gathers(0, 0)
        sd = [None, None]
        for ci in range(nchunk):
            b = ci % 2
            for d in gd:
                d.wait()
            if ci + 1 < nchunk:
                gd = issue_gathers(ci + 1, 1 - b)
            if sd[b] is not None:
                for d in sd[b]:
                    d.wait()
            compute(ci, b, ebase)
            sd[b] = issue_scatters(ci, b)
        for dl in sd:
            if dl is not None:
                for d in dl:
                    d.wait()

    @pl.when(cid == 0)
    def _():
        run_pipeline(sid * EPW0, EPW0 // CH)

    @pl.when(cid == 1)
    def _():
        run_pipeline(NS * EPW0 + sid * EPW1, EPW1 // CH)

    plsc.subcore_barrier()

    row0 = sid * ROWS_PER_SUB
    pltpu.sync_copy(acc_num.at[pl.ds(row0, ROWS_PER_SUB)],
                    num_o.at[cid, pl.ds(row0, ROWS_PER_SUB)])
    pltpu.sync_copy(acc_den.at[pl.ds(row0, ROWS_PER_SUB)],
                    den_o.at[cid, pl.ds(row0, ROWS_PER_SUB)])


_edge_pass = pl.kernel(
    _edge_pass_body,
    out_type=(
        jax.ShapeDtypeStruct((NC, NPAD, HH), _f32),
        jax.ShapeDtypeStruct((NC, NPAD), _f32),
    ),
    compiler_params=_SC_PARAMS,
    mesh=plsc.VectorSubcoreMesh(
        core_axis_name="c", subcore_axis_name="s", num_cores=NC,
        num_subcores=NS),
    scratch_types=[
        pltpu.VMEM((EPW0,), jnp.int32),
        pltpu.VMEM((EPW0,), jnp.int32),
        pltpu.VMEM((EPW0,), jnp.int32),
        pltpu.VMEM((2, CH), _f32),
        pltpu.VMEM((2, CH), _f32),
        pltpu.VMEM((2, CH), _f32),
        pltpu.VMEM((2, CH, HH), _f32),
        pltpu.VMEM((2, CH, HH), _f32),
        pltpu.VMEM_SHARED((NPAD, HH), _f32),
        pltpu.VMEM_SHARED((NPAD,), _f32),
        pltpu.SemaphoreType.DMA,
        pltpu.SemaphoreType.DMA,
    ],
)


def _tables_body(ne_ref, pwt_ref, pb_ref, ee_ref,
                 w11_ref, b11_ref, w12_ref, b12_ref,
                 w21_ref, b21_ref, w22_ref, b22_ref,
                 q_ref, t_ref):
    q_ref[...] = jnp.maximum(
        jnp.dot(ne_ref[...], pwt_ref[...], preferred_element_type=_f32)
        + pb_ref[...], 0.0)
    ee = ee_ref[...]
    t1 = jnp.maximum(
        jnp.dot(ee, w11_ref[...], preferred_element_type=_f32) + b11_ref[...],
        0.0)
    t1 = jnp.dot(t1, w12_ref[...], preferred_element_type=_f32) + b12_ref[...]
    t2 = jnp.maximum(
        jnp.dot(ee, w21_ref[...], preferred_element_type=_f32) + b21_ref[...],
        0.0)
    t2 = jnp.dot(t2, w22_ref[...], preferred_element_type=_f32) + b22_ref[...]
    tid = lax.broadcasted_iota(jnp.int32, (NTYPE, 1), 0)
    t_ref[...] = jnp.where(tid < 4, t1, t2)


def _tc_tables(node_emb, proj_wt, proj_b, edge_emb,
               w11, b11, w12, b12, w21, b21, w22, b22):
    return pl.pallas_call(
        _tables_body,
        out_shape=(
            jax.ShapeDtypeStruct((NN, HH), _f32),
            jax.ShapeDtypeStruct((NTYPE, HH * HH), _f32),
        ),
    )(node_emb, proj_wt, proj_b, edge_emb,
      w11, b11, w12, b12, w21, b21, w22, b22)


def _gru(hn, h, gw):
    (wir, wiz, win, whr, whz, whn, bir, biz, bin_, bhr, bhz, bhn) = gw
    ir = jnp.dot(hn, wir, preferred_element_type=_f32) + bir
    iz = jnp.dot(hn, wiz, preferred_element_type=_f32) + biz
    inn = jnp.dot(hn, win, preferred_element_type=_f32) + bin_
    hr = jnp.dot(h, whr, preferred_element_type=_f32) + bhr
    hz = jnp.dot(h, whz, preferred_element_type=_f32) + bhz
    hnn = jnp.dot(h, whn, preferred_element_type=_f32) + bhn
    r = jax.nn.sigmoid(ir + hr)
    z = jax.nn.sigmoid(iz + hz)
    n = jnp.tanh(inn + r * hnn)
    return (1.0 - z) * n + z * h


def _prep_out(hnew, wcat_ref, aw1_ref, aw2_ref, g_ref, a1_ref, a2_ref):
    g_ref[...] = jnp.dot(hnew, wcat_ref[...], preferred_element_type=_f32)
    a1_ref[...] = jnp.sum(hnew * aw1_ref[...], axis=1, keepdims=True)
    a2_ref[...] = jnp.sum(hnew * aw2_ref[...], axis=1, keepdims=True)


NBLK = 10
BLK = NPAD // NBLK


def _step_body(num_ref, den_ref, h_ref, convb_ref,
               wir, wiz, win, whr, whz, whn, bir, biz, bin_, bhr, bhz, bhn,
               wcat_ref, aw1_ref, aw2_ref,
               hnew_ref, g_ref, a1_ref, a2_ref):
    num = num_ref[0] + num_ref[1]
    den = den_ref[0] + den_ref[1]
    agg = num / (den + 1e-16) + convb_ref[...]
    hn = jnp.maximum(agg, 0.0)
    h = h_ref[...]
    gw = (wir[...], wiz[...], win[...], whr[...], whz[...], whn[...],
          bir[...], biz[...], bin_[...], bhr[...], bhz[...], bhn[...])
    hnew = _gru(hn, h, gw)
    hnew_ref[...] = hnew
    _prep_out(hnew, wcat_ref, aw1_ref, aw2_ref, g_ref, a1_ref, a2_ref)


def _final_body(num_ref, den_ref, h_ref, convb_ref,
                wir, wiz, win, whr, whz, whn, bir, biz, bin_, bhr, bhz, bhn,
                hnew_ref):
    num = num_ref[0] + num_ref[1]
    den = den_ref[0] + den_ref[1]
    agg = num / (den + 1e-16) + convb_ref[...]
    hn = jnp.maximum(agg, 0.0)
    h = h_ref[...]
    gw = (wir[...], wiz[...], win[...], whr[...], whz[...], whn[...],
          bir[...], biz[...], bin_[...], bhr[...], bhz[...], bhn[...])
    hnew_ref[...] = _gru(hn, h, gw)


def _prep_body(h_ref, wcat_ref, aw1_ref, aw2_ref, g_ref, a1_ref, a2_ref):
    _prep_out(h_ref[...], wcat_ref, aw1_ref, aw2_ref, g_ref, a1_ref, a2_ref)


def _blk(shape):
    return pl.BlockSpec(shape, lambda i: (0,) * len(shape))


_num_spec = pl.BlockSpec((NC, BLK, HH), lambda i: (0, i, 0))
_den_spec = pl.BlockSpec((NC, BLK, 1), lambda i: (0, i, 0))
_h_spec = pl.BlockSpec((BLK, HH), lambda i: (i, 0))
_g_spec = pl.BlockSpec((BLK, NTYPE * HH), lambda i: (i, 0))
_a_spec = pl.BlockSpec((BLK, 1), lambda i: (i, 0))

_GW_SHAPES = [(HH, HH)] * 6 + [(1, HH)] * 6


def _tc_step(num, den, h, convb, gws, wcat, aw1, aw2):
    return pl.pallas_call(
        _step_body,
        grid=(NBLK,),
        in_specs=[_num_spec, _den_spec, _h_spec, _blk((1, HH))] +
                 [_blk(s) for s in _GW_SHAPES] +
                 [_blk((HH, NTYPE * HH)), _blk((1, HH)), _blk((1, HH))],
        out_specs=[_h_spec, _g_spec, _a_spec, _a_spec],
        out_shape=(
            jax.ShapeDtypeStruct((NPAD, HH), _f32),
            jax.ShapeDtypeStruct((NPAD, NTYPE * HH), _f32),
            jax.ShapeDtypeStruct((NPAD, 1), _f32),
            jax.ShapeDtypeStruct((NPAD, 1), _f32),
        ),
    )(num, den, h, convb, *gws, wcat, aw1, aw2)


def _tc_final(num, den, h, convb, gws):
    return pl.pallas_call(
        _final_body,
        grid=(NBLK,),
        in_specs=[_num_spec, _den_spec, _h_spec, _blk((1, HH))] +
                 [_blk(s) for s in _GW_SHAPES],
        out_specs=[_h_spec],
        out_shape=(jax.ShapeDtypeStruct((NPAD, HH), _f32),),
    )(num, den, h, convb, *gws)[0]


def _tc_prep(h, wcat, aw1, aw2):
    return pl.pallas_call(
        _prep_body,
        grid=(NBLK,),
        in_specs=[_h_spec, _blk((HH, NTYPE * HH)), _blk((1, HH)),
                  _blk((1, HH))],
        out_specs=[_g_spec, _a_spec, _a_spec],
        out_shape=(
            jax.ShapeDtypeStruct((NPAD, NTYPE * HH), _f32),
            jax.ShapeDtypeStruct((NPAD, 1), _f32),
            jax.ShapeDtypeStruct((NPAD, 1), _f32),
        ),
    )(h, wcat, aw1, aw2)


@jax.jit
def kernel(node_ids, edge_ids, edge_index, node_emb, edge_emb, proj_w, proj_b,
           attn_w, e1_w1, e1_b1, e1_w2, e1_b2, e2_w1, e2_b1, e2_w2, e2_b2,
           conv_b, gru_w_ih, gru_w_hh, gru_b_ih, gru_b_hh):
    src = edge_index[0].astype(jnp.int32)
    dst = edge_index[1].astype(jnp.int32)
    et = edge_ids.astype(jnp.int32)
    pad = EPAD - EE
    srcp = jnp.concatenate([src, jnp.zeros((pad,), jnp.int32)])
    dstp = jnp.concatenate([dst, jnp.zeros((pad,), jnp.int32)])
    gidxp = jnp.concatenate([src * NTYPE + et, jnp.zeros((pad,), jnp.int32)])
    nidp = jnp.concatenate(
        [node_ids.astype(jnp.int32), jnp.zeros((NPAD - NN,), jnp.int32)])

    convb = conv_b.reshape(1, HH)
    aw1 = attn_w[:, :HH].reshape(1, HH)
    aw2 = attn_w[:, HH:].reshape(1, HH)
    gws = (gru_w_ih[:HH].T, gru_w_ih[HH:2 * HH].T, gru_w_ih[2 * HH:].T,
           gru_w_hh[:HH].T, gru_w_hh[HH:2 * HH].T, gru_w_hh[2 * HH:].T,
           gru_b_ih[:HH].reshape(1, HH), gru_b_ih[HH:2 * HH].reshape(1, HH),
           gru_b_ih[2 * HH:].reshape(1, HH),
           gru_b_hh[:HH].reshape(1, HH), gru_b_hh[HH:2 * HH].reshape(1, HH),
           gru_b_hh[2 * HH:].reshape(1, HH))

    q, t = _tc_tables(node_emb, proj_w.T, proj_b.reshape(1, HH), edge_emb,
                      e1_w1.T, e1_b1.reshape(1, -1), e1_w2.T,
                      e1_b2.reshape(1, -1), e2_w1.T, e2_b1.reshape(1, -1),
                      e2_w2.T, e2_b2.reshape(1, -1))
    wcat = t.reshape(NTYPE, HH, HH).transpose(1, 0, 2).reshape(HH, NTYPE * HH)

    h = _h0_gather(nidp, q)
    g, a1, a2 = _tc_prep(h, wcat, aw1, aw2)

    for step in range(3):
        gflat = g.reshape(NNP, HH)
        num_p, den_p = _edge_pass(srcp, dstp, gidxp, a1.reshape(NPAD),
                                  a2.reshape(NPAD), gflat)
        den = den_p.reshape(NC, NPAD, 1)
        if step < 2:
            h, g, a1, a2 = _tc_step(num_p, den, h, convb, gws, wcat, aw1, aw2)
        else:
            h = _tc_final(num_p, den, h, convb, gws)
    return h[:NN]


# speedup vs baseline: 1.5713x; 1.0007x over previous
import jax
import jax.numpy as jnp
from jax import lax
from jax.experimental import pallas as pl
from jax.experimental.pallas import tpu as pltpu
from jax.experimental.pallas import tpu_sc as plsc

NN = 10000
EE = 160000
HH = 16
NTYPE = 50
NC, NS, LANES = 2, 16, 16
NW = NC * NS
CH = 1024
EPW0 = 6 * CH
EPW1 = 4 * CH
EPAD = (EPW0 + EPW1) * NS
NPAD = 10240
NPW = NPAD // NW
NNP = NPAD * NTYPE
ROWS_PER_SUB = NPAD // NS
AW = 24

_f32 = jnp.float32

_SC_PARAMS = pltpu.CompilerParams(use_tc_tiling_on_sc=False,
                                  needs_layout_passes=False)


def _h0_gather_body(nid_h, q_h, out_h, idxv, rowsv, sem):
    cid = lax.axis_index("c")
    sid = lax.axis_index("s")
    wid = sid * NC + cid
    for c in range(NPW // 80):
        base = wid * NPW + c * 80
        pltpu.sync_copy(nid_h.at[pl.ds(base, 80)], idxv)
        pltpu.async_copy(q_h.at[idxv], rowsv, sem).wait()
        pltpu.sync_copy(rowsv, out_h.at[pl.ds(base, 80)])


_h0_gather = pl.kernel(
    _h0_gather_body,
    out_type=jax.ShapeDtypeStruct((NPAD, HH), _f32),
    compiler_params=_SC_PARAMS,
    mesh=plsc.VectorSubcoreMesh(
        core_axis_name="c", subcore_axis_name="s", num_cores=NC,
        num_subcores=NS),
    scratch_types=[
        pltpu.VMEM((80,), jnp.int32),
        pltpu.VMEM((80, HH), _f32),
        pltpu.SemaphoreType.DMA,
    ],
)


def _edge_pass_body(src_h, dst_h, gidx_h, a1_h, a2_h, g_h,
                    acc_o,
                    srcv, dstv, gidxv, a1v, a2v, gv, outv,
                    acc_num, sem, ssem):
    cid = lax.axis_index("c")
    sid = lax.axis_index("s")

    zero16 = jnp.zeros((LANES,), _f32)
    for b in range(2):
        for i in range(CH):
            outv[b, i, pl.ds(0, LANES)] = zero16
            outv[b, i, pl.ds(AW - LANES, LANES)] = zero16
    pltpu.sync_copy(outv.at[0, pl.ds(0, ROWS_PER_SUB)],
                    acc_num.at[pl.ds(sid * ROWS_PER_SUB, ROWS_PER_SUB)])
    plsc.subcore_barrier()

    lane_iota = lax.iota(jnp.int32, LANES)
    col16 = jnp.full((LANES,), HH, jnp.int32)

    def issue_gathers(ci, b):
        c0 = ci * CH
        return [
            pltpu.async_copy(a1_h.at[srcv.at[pl.ds(c0, CH)]], a1v.at[b], sem),
            pltpu.async_copy(a2_h.at[dstv.at[pl.ds(c0, CH)]], a2v.at[b], sem),
            pltpu.async_copy(g_h.at[gidxv.at[pl.ds(c0, CH)]], gv.at[b], sem),
        ]

    def compute(ci, b, ebase):
        base = ebase + ci * CH

        def group(gi, c):
            rowi = gi * LANES + lane_iota
            s = (plsc.load_gather(a1v.at[b], [rowi])
                 + plsc.load_gather(a2v.at[b], [rowi]))
            l = jnp.where(s >= 0.0, s, 0.2 * s)
            ex = jnp.exp(l)
            eid = base + rowi
            ex = jnp.where(eid < EE, ex, 0.0)
            plsc.store_scatter(outv.at[b], [rowi, col16], ex)
            for j in range(HH):
                colj = jnp.full((LANES,), j, jnp.int32)
                colt = plsc.load_gather(gv.at[b], [rowi, colj])
                plsc.store_scatter(outv.at[b], [rowi, colj], colt * ex)
            return c

        lax.fori_loop(0, CH // LANES, group, 0)

    def issue_scatters(ci, b):
        c0 = ci * CH
        return [
            pltpu.async_copy(outv.at[b], acc_num.at[dstv.at[pl.ds(c0, CH)]],
                             ssem, add=True),
        ]

    def run_pipeline(ebase, nchunk):
        epw = nchunk * CH
        pltpu.sync_copy(src_h.at[pl.ds(ebase, epw)], srcv.at[pl.ds(0, epw)])
        pltpu.sync_copy(dst_h.at[pl.ds(ebase, epw)], dstv.at[pl.ds(0, epw)])
        pltpu.sync_copy(gidx_h.at[pl.ds(ebase, epw)], gidxv.at[pl.ds(0, epw)])
        gd = issue_gathers(0, 0)
        sd = [None, None]
        for ci in range(nchunk):
            b = ci % 2
            for d in gd:
                d.wait()
            if ci + 1 < nchunk:
                gd = issue_gathers(ci + 1, 1 - b)
            if sd[b] is not None:
                for d in sd[b]:
                    d.wait()
            compute(ci, b, ebase)
            sd[b] = issue_scatters(ci, b)
        for dl in sd:
            if dl is not None:
                for d in dl:
                    d.wait()

    @pl.when(cid == 0)
    def _():
        run_pipeline(sid * EPW0, EPW0 // CH)

    @pl.when(cid == 1)
    def _():
        run_pipeline(NS * EPW0 + sid * EPW1, EPW1 // CH)

    plsc.subcore_barrier()

    row0 = sid * ROWS_PER_SUB
    pltpu.sync_copy(acc_num.at[pl.ds(row0, ROWS_PER_SUB)],
                    acc_o.at[cid, pl.ds(row0, ROWS_PER_SUB)])


_edge_pass = pl.kernel(
    _edge_pass_body,
    out_type=jax.ShapeDtypeStruct((NC, NPAD, AW), _f32),
    compiler_params=_SC_PARAMS,
    mesh=plsc.VectorSubcoreMesh(
        core_axis_name="c", subcore_axis_name="s", num_cores=NC,
        num_subcores=NS),
    scratch_types=[
        pltpu.VMEM((EPW0,), jnp.int32),
        pltpu.VMEM((EPW0,), jnp.int32),
        pltpu.VMEM((EPW0,), jnp.int32),
        pltpu.VMEM((2, CH), _f32),
        pltpu.VMEM((2, CH), _f32),
        pltpu.VMEM((2, CH, HH), _f32),
        pltpu.VMEM((2, CH, AW), _f32),
        pltpu.VMEM_SHARED((NPAD, AW), _f32),
        pltpu.SemaphoreType.DMA,
        pltpu.SemaphoreType.DMA,
    ],
)


def _tables_body(ne_ref, pwt_ref, pb_ref, ee_ref,
                 w11_ref, b11_ref, w12_ref, b12_ref,
                 w21_ref, b21_ref, w22_ref, b22_ref,
                 q_ref, t_ref):
    q_ref[...] = jnp.maximum(
        jnp.dot(ne_ref[...], pwt_ref[...], preferred_element_type=_f32)
        + pb_ref[...], 0.0)
    ee = ee_ref[...]
    t1 = jnp.maximum(
        jnp.dot(ee, w11_ref[...], preferred_element_type=_f32) + b11_ref[...],
        0.0)
    t1 = jnp.dot(t1, w12_ref[...], preferred_element_type=_f32) + b12_ref[...]
    t2 = jnp.maximum(
        jnp.dot(ee, w21_ref[...], preferred_element_type=_f32) + b21_ref[...],
        0.0)
    t2 = jnp.dot(t2, w22_ref[...], preferred_element_type=_f32) + b22_ref[...]
    tid = lax.broadcasted_iota(jnp.int32, (NTYPE, 1), 0)
    t_ref[...] = jnp.where(tid < 4, t1, t2)


def _tc_tables(node_emb, proj_wt, proj_b, edge_emb,
               w11, b11, w12, b12, w21, b21, w22, b22):
    return pl.pallas_call(
        _tables_body,
        out_shape=(
            jax.ShapeDtypeStruct((NN, HH), _f32),
            jax.ShapeDtypeStruct((NTYPE, HH * HH), _f32),
        ),
    )(node_emb, proj_wt, proj_b, edge_emb,
      w11, b11, w12, b12, w21, b21, w22, b22)


def _gru(hn, h, gw):
    (wir, wiz, win, whr, whz, whn, bir, biz, bin_, bhr, bhz, bhn) = gw
    ir = jnp.dot(hn, wir, preferred_element_type=_f32) + bir
    iz = jnp.dot(hn, wiz, preferred_element_type=_f32) + biz
    inn = jnp.dot(hn, win, preferred_element_type=_f32) + bin_
    hr = jnp.dot(h, whr, preferred_element_type=_f32) + bhr
    hz = jnp.dot(h, whz, preferred_element_type=_f32) + bhz
    hnn = jnp.dot(h, whn, preferred_element_type=_f32) + bhn
    r = jax.nn.sigmoid(ir + hr)
    z = jax.nn.sigmoid(iz + hz)
    n = jnp.tanh(inn + r * hnn)
    return (1.0 - z) * n + z * h


def _prep_out(hnew, wcat_ref, aw1_ref, aw2_ref, g_ref, a1_ref, a2_ref):
    g_ref[...] = jnp.dot(hnew, wcat_ref[...], preferred_element_type=_f32)
    a1_ref[...] = jnp.sum(hnew * aw1_ref[...], axis=1, keepdims=True)
    a2_ref[...] = jnp.sum(hnew * aw2_ref[...], axis=1, keepdims=True)


NBLK = 10
BLK = NPAD // NBLK


def _step_body(num_ref, h_ref, convb_ref,
               wir, wiz, win, whr, whz, whn, bir, biz, bin_, bhr, bhz, bhn,
               wcat_ref, aw1_ref, aw2_ref,
               hnew_ref, g_ref, a1_ref, a2_ref):
    acc = num_ref[0] + num_ref[1]
    num = acc[:, :HH]
    den = acc[:, HH:HH + 1]
    agg = num / (den + 1e-16) + convb_ref[...]
    hn = jnp.maximum(agg, 0.0)
    h = h_ref[...]
    gw = (wir[...], wiz[...], win[...], whr[...], whz[...], whn[...],
          bir[...], biz[...], bin_[...], bhr[...], bhz[...], bhn[...])
    hnew = _gru(hn, h, gw)
    hnew_ref[...] = hnew
    _prep_out(hnew, wcat_ref, aw1_ref, aw2_ref, g_ref, a1_ref, a2_ref)


def _final_body(num_ref, h_ref, convb_ref,
                wir, wiz, win, whr, whz, whn, bir, biz, bin_, bhr, bhz, bhn,
                hnew_ref):
    acc = num_ref[0] + num_ref[1]
    num = acc[:, :HH]
    den = acc[:, HH:HH + 1]
    agg = num / (den + 1e-16) + convb_ref[...]
    hn = jnp.maximum(agg, 0.0)
    h = h_ref[...]
    gw = (wir[...], wiz[...], win[...], whr[...], whz[...], whn[...],
          bir[...], biz[...], bin_[...], bhr[...], bhz[...], bhn[...])
    hnew_ref[...] = _gru(hn, h, gw)


def _prep_body(h_ref, wcat_ref, aw1_ref, aw2_ref, g_ref, a1_ref, a2_ref):
    _prep_out(h_ref[...], wcat_ref, aw1_ref, aw2_ref, g_ref, a1_ref, a2_ref)


def _blk(shape):
    return pl.BlockSpec(shape, lambda i: (0,) * len(shape))


_num_spec = pl.BlockSpec((NC, BLK, AW), lambda i: (0, i, 0))
_h_spec = pl.BlockSpec((BLK, HH), lambda i: (i, 0))
_g_spec = pl.BlockSpec((BLK, NTYPE * HH), lambda i: (i, 0))
_a_spec = pl.BlockSpec((BLK, 1), lambda i: (i, 0))

_GW_SHAPES = [(HH, HH)] * 6 + [(1, HH)] * 6


def _tc_step(num, h, convb, gws, wcat, aw1, aw2):
    return pl.pallas_call(
        _step_body,
        grid=(NBLK,),
        in_specs=[_num_spec, _h_spec, _blk((1, HH))] +
                 [_blk(s) for s in _GW_SHAPES] +
                 [_blk((HH, NTYPE * HH)), _blk((1, HH)), _blk((1, HH))],
        out_specs=[_h_spec, _g_spec, _a_spec, _a_spec],
        out_shape=(
            jax.ShapeDtypeStruct((NPAD, HH), _f32),
            jax.ShapeDtypeStruct((NPAD, NTYPE * HH), _f32),
            jax.ShapeDtypeStruct((NPAD, 1), _f32),
            jax.ShapeDtypeStruct((NPAD, 1), _f32),
        ),
    )(num, h, convb, *gws, wcat, aw1, aw2)


def _tc_final(num, h, convb, gws):
    return pl.pallas_call(
        _final_body,
        grid=(NBLK,),
        in_specs=[_num_spec, _h_spec, _blk((1, HH))] +
                 [_blk(s) for s in _GW_SHAPES],
        out_specs=[_h_spec],
        out_shape=(jax.ShapeDtypeStruct((NPAD, HH), _f32),),
    )(num, h, convb, *gws)[0]


def _tc_prep(h, wcat, aw1, aw2):
    return pl.pallas_call(
        _prep_body,
        grid=(NBLK,),
        in_specs=[_h_spec, _blk((HH, NTYPE * HH)), _blk((1, HH)),
                  _blk((1, HH))],
        out_specs=[_g_spec, _a_spec, _a_spec],
        out_shape=(
            jax.ShapeDtypeStruct((NPAD, NTYPE * HH), _f32),
            jax.ShapeDtypeStruct((NPAD, 1), _f32),
            jax.ShapeDtypeStruct((NPAD, 1), _f32),
        ),
    )(h, wcat, aw1, aw2)


@jax.jit
def kernel(node_ids, edge_ids, edge_index, node_emb, edge_emb, proj_w, proj_b,
           attn_w, e1_w1, e1_b1, e1_w2, e1_b2, e2_w1, e2_b1, e2_w2, e2_b2,
           conv_b, gru_w_ih, gru_w_hh, gru_b_ih, gru_b_hh):
    src = edge_index[0].astype(jnp.int32)
    dst = edge_index[1].astype(jnp.int32)
    et = edge_ids.astype(jnp.int32)
    pad = EPAD - EE
    srcp = jnp.concatenate([src, jnp.zeros((pad,), jnp.int32)])
    dstp = jnp.concatenate([dst, jnp.zeros((pad,), jnp.int32)])
    gidxp = jnp.concatenate([src * NTYPE + et, jnp.zeros((pad,), jnp.int32)])
    nidp = jnp.concatenate(
        [node_ids.astype(jnp.int32), jnp.zeros((NPAD - NN,), jnp.int32)])

    convb = conv_b.reshape(1, HH)
    aw1 = attn_w[:, :HH].reshape(1, HH)
    aw2 = attn_w[:, HH:].reshape(1, HH)
    gws = (gru_w_ih[:HH].T, gru_w_ih[HH:2 * HH].T, gru_w_ih[2 * HH:].T,
           gru_w_hh[:HH].T, gru_w_hh[HH:2 * HH].T, gru_w_hh[2 * HH:].T,
           gru_b_ih[:HH].reshape(1, HH), gru_b_ih[HH:2 * HH].reshape(1, HH),
           gru_b_ih[2 * HH:].reshape(1, HH),
           gru_b_hh[:HH].reshape(1, HH), gru_b_hh[HH:2 * HH].reshape(1, HH),
           gru_b_hh[2 * HH:].reshape(1, HH))

    q, t = _tc_tables(node_emb, proj_w.T, proj_b.reshape(1, HH), edge_emb,
                      e1_w1.T, e1_b1.reshape(1, -1), e1_w2.T,
                      e1_b2.reshape(1, -1), e2_w1.T, e2_b1.reshape(1, -1),
                      e2_w2.T, e2_b2.reshape(1, -1))
    wcat = t.reshape(NTYPE, HH, HH).transpose(1, 0, 2).reshape(HH, NTYPE * HH)

    h = _h0_gather(nidp, q)
    g, a1, a2 = _tc_prep(h, wcat, aw1, aw2)

    for step in range(3):
        gflat = g.reshape(NNP, HH)
        acc = _edge_pass(srcp, dstp, gidxp, a1.reshape(NPAD),
                         a2.reshape(NPAD), gflat)
        if step < 2:
            h, g, a1, a2 = _tc_step(acc, h, convb, gws, wcat, aw1, aw2)
        else:
            h = _tc_final(acc, h, convb, gws)
    return h[:NN]
